# Initial kernel scaffold; baseline (speedup 1.0000x reference)
import jax
import jax.numpy as jnp
from jax.experimental import pallas as pl


def kernel(h, e_h, norm, edge_index, W, b):
    raise NotImplementedError("write your pallas kernel here")


# speedup vs baseline: 40.3917x; 40.3917x over previous
import dataclasses

import jax
import jax.numpy as jnp
from jax import lax
from jax.experimental import pallas as pl
from jax.experimental.pallas import tpu as pltpu
from jax.experimental.pallas import tpu_sc as plsc

_N = 10000
_E = 320000
_D_IN = 128
_D_OUT = 128

_NUM_CORES = 2
_NUM_SUBCORES = 16
_NW = _NUM_CORES * _NUM_SUBCORES
_LANES = 16
_EDGE_BLOCK = 1280


def _sc_segment_sum(dst2d, ev2d):
    mesh = plsc.VectorSubcoreMesh(core_axis_name="c", subcore_axis_name="s")
    cp = pltpu.CompilerParams()
    if "needs_layout_passes" in pltpu.CompilerParams.__dataclass_fields__:
        cp = dataclasses.replace(cp, needs_layout_passes=False)

    @pl.kernel(
        out_type=jax.ShapeDtypeStruct((_NW, _N), jnp.float32),
        mesh=mesh,
        scratch_types=[pltpu.VMEM((_N,), jnp.float32)],
        compiler_params=cp,
    )
    def seg_sum_kernel(dst_hbm, ev_hbm, out_hbm, acc):
        @pl.loop(0, _N, step=_LANES)
        def _(i):
            acc.at[pl.ds(i, _LANES)][...] = jnp.zeros((_LANES,), jnp.float32)

        def body(i_vmem, v_vmem):
            @pl.loop(0, _EDGE_BLOCK, step=_LANES)
            def _(c):
                idx = i_vmem.at[0, pl.ds(c, _LANES)][...]
                val = v_vmem.at[0, pl.ds(c, _LANES)][...]
                plsc.addupdate_scatter(acc, [idx], val)

        pltpu.emit_pipeline(
            body,
            grid=(_E // _EDGE_BLOCK,),
            in_specs=[
                pl.BlockSpec((1, _EDGE_BLOCK), lambda i: (0, i)),
                pl.BlockSpec((1, _EDGE_BLOCK), lambda i: (0, i)),
            ],
            out_specs=[],
            core_axis_name=("c", "s"),
            dimension_semantics=(pltpu.PARALLEL,),
        )(dst_hbm, ev_hbm)

        wid = lax.axis_index("s") * _NUM_CORES + lax.axis_index("c")
        pltpu.sync_copy(acc, out_hbm.at[wid])

    return seg_sum_kernel(dst2d, ev2d)


_ROWS = 1000


def _tc_body(h_ref, sp_ref, norm_ref, wt_ref, b_ref, o_ref):
    s = jnp.sum(sp_ref[...], axis=1, keepdims=True)
    x = h_ref[...] * s
    y = jnp.dot(x, wt_ref[...], preferred_element_type=jnp.float32)
    o_ref[...] = (y + b_ref[...]) * norm_ref[...]


def _tc_apply(h, s_part_t, norm, wt, b2d):
    return pl.pallas_call(
        _tc_body,
        grid=(_N // _ROWS,),
        in_specs=[
            pl.BlockSpec((_ROWS, _D_IN), lambda i: (i, 0)),
            pl.BlockSpec((_ROWS, _NW), lambda i: (i, 0)),
            pl.BlockSpec((_ROWS, 1), lambda i: (i, 0)),
            pl.BlockSpec((_D_IN, _D_OUT), lambda i: (0, 0)),
            pl.BlockSpec((1, _D_OUT), lambda i: (0, 0)),
        ],
        out_specs=pl.BlockSpec((_ROWS, _D_OUT), lambda i: (i, 0)),
        out_shape=jax.ShapeDtypeStruct((_N, _D_OUT), jnp.float32),
    )(h, s_part_t, norm, wt, b2d)


def kernel(h, e_h, norm, edge_index, W, b):
    dst = edge_index[1].reshape(1, _E)
    ev = e_h.reshape(1, _E)
    s_part = _sc_segment_sum(dst, ev)
    return _tc_apply(h, s_part.T, norm, W.T, b.reshape(1, _D_OUT))


# speedup vs baseline: 41.9395x; 1.0383x over previous
import dataclasses

import jax
import jax.numpy as jnp
from jax import lax
from jax.experimental import pallas as pl
from jax.experimental.pallas import tpu as pltpu
from jax.experimental.pallas import tpu_sc as plsc

_N = 10000
_E = 320000
_D_IN = 128
_D_OUT = 128

_NUM_CORES = 2
_NUM_SUBCORES = 16
_NW = _NUM_CORES * _NUM_SUBCORES
_LANES = 16
_EDGE_BLOCK = 1280


def _sc_segment_sum(dst2d, ev2d):
    mesh = plsc.VectorSubcoreMesh(core_axis_name="c", subcore_axis_name="s")
    cp = pltpu.CompilerParams()
    if "needs_layout_passes" in pltpu.CompilerParams.__dataclass_fields__:
        cp = dataclasses.replace(cp, needs_layout_passes=False)

    @pl.kernel(
        out_type=jax.ShapeDtypeStruct((_NW, _N), jnp.float32),
        mesh=mesh,
        scratch_types=[pltpu.VMEM((_N,), jnp.float32)],
        compiler_params=cp,
    )
    def seg_sum_kernel(dst_hbm, ev_hbm, out_hbm, acc):
        @pl.loop(0, _N, step=_LANES, unroll=8)
        def _(i):
            acc.at[pl.ds(i, _LANES)][...] = jnp.zeros((_LANES,), jnp.float32)

        def body(i_vmem, v_vmem):
            @pl.loop(0, _EDGE_BLOCK, step=_LANES, unroll=8)
            def _(c):
                idx = i_vmem.at[0, pl.ds(c, _LANES)][...]
                val = v_vmem.at[0, pl.ds(c, _LANES)][...]
                plsc.addupdate_scatter(acc, [idx], val)

        pltpu.emit_pipeline(
            body,
            grid=(_E // _EDGE_BLOCK,),
            in_specs=[
                pl.BlockSpec((1, _EDGE_BLOCK), lambda i: (0, i)),
                pl.BlockSpec((1, _EDGE_BLOCK), lambda i: (0, i)),
            ],
            out_specs=[],
            core_axis_name=("c", "s"),
            dimension_semantics=(pltpu.PARALLEL,),
        )(dst_hbm, ev_hbm)

        wid = lax.axis_index("s") * _NUM_CORES + lax.axis_index("c")
        pltpu.sync_copy(acc, out_hbm.at[wid])

    return seg_sum_kernel(dst2d, ev2d)


_ROWS = 1000


def _tc_body(h_ref, sp_ref, norm_ref, wt_ref, b_ref, o_ref):
    s = jnp.sum(sp_ref[...], axis=1, keepdims=True)
    x = h_ref[...] * s
    y = jnp.dot(x, wt_ref[...], preferred_element_type=jnp.float32)
    o_ref[...] = (y + b_ref[...]) * norm_ref[...]


def _tc_apply(h, s_part_t, norm, wt, b2d):
    return pl.pallas_call(
        _tc_body,
        grid=(_N // _ROWS,),
        in_specs=[
            pl.BlockSpec((_ROWS, _D_IN), lambda i: (i, 0)),
            pl.BlockSpec((_ROWS, _NW), lambda i: (i, 0)),
            pl.BlockSpec((_ROWS, 1), lambda i: (i, 0)),
            pl.BlockSpec((_D_IN, _D_OUT), lambda i: (0, 0)),
            pl.BlockSpec((1, _D_OUT), lambda i: (0, 0)),
        ],
        out_specs=pl.BlockSpec((_ROWS, _D_OUT), lambda i: (i, 0)),
        out_shape=jax.ShapeDtypeStruct((_N, _D_OUT), jnp.float32),
    )(h, s_part_t, norm, wt, b2d)


def kernel(h, e_h, norm, edge_index, W, b):
    dst = edge_index[1].reshape(1, _E)
    ev = e_h.reshape(1, _E)
    s_part = _sc_segment_sum(dst, ev)
    return _tc_apply(h, s_part.T, norm, W.T, b.reshape(1, _D_OUT))


# speedup vs baseline: 50.7880x; 1.2110x over previous
import dataclasses

import jax
import jax.numpy as jnp
from jax import lax
from jax.experimental import pallas as pl
from jax.experimental.pallas import tpu as pltpu
from jax.experimental.pallas import tpu_sc as plsc

_N = 10000
_E = 320000
_D_IN = 128
_D_OUT = 128

_NUM_CORES = 2
_NUM_SUBCORES = 16
_NW = _NUM_CORES * _NUM_SUBCORES
_LANES = 16
_EDGE_BLOCK = 1280


def _sc_segment_sum(ei2d, ev2d):
    mesh = plsc.VectorSubcoreMesh(core_axis_name="c", subcore_axis_name="s")
    cp = pltpu.CompilerParams()
    if "needs_layout_passes" in pltpu.CompilerParams.__dataclass_fields__:
        cp = dataclasses.replace(cp, needs_layout_passes=False)

    @pl.kernel(
        out_type=jax.ShapeDtypeStruct((_NW, _N), jnp.float32),
        mesh=mesh,
        scratch_types=[pltpu.VMEM((_N,), jnp.float32)],
        compiler_params=cp,
    )
    def seg_sum_kernel(ei_hbm, ev_hbm, out_hbm, acc):
        @pl.loop(0, _N, step=_LANES, unroll=8)
        def _(i):
            acc.at[pl.ds(i, _LANES)][...] = jnp.zeros((_LANES,), jnp.float32)

        def body(i_vmem, v_vmem):
            @pl.loop(0, _EDGE_BLOCK, step=_LANES, unroll=8)
            def _(c):
                idx = i_vmem.at[0, pl.ds(c, _LANES)][...]
                val = v_vmem.at[0, pl.ds(c, _LANES)][...]
                plsc.addupdate_scatter(acc, [idx], val)

        pltpu.emit_pipeline(
            body,
            grid=(_E // _EDGE_BLOCK,),
            in_specs=[
                pl.BlockSpec((1, _EDGE_BLOCK), lambda i: (1, i)),
                pl.BlockSpec((1, _EDGE_BLOCK), lambda i: (0, i)),
            ],
            out_specs=[],
            core_axis_name=("c", "s"),
            dimension_semantics=(pltpu.PARALLEL,),
        )(ei_hbm, ev_hbm)

        wid = lax.axis_index("s") * _NUM_CORES + lax.axis_index("c")
        pltpu.sync_copy(acc, out_hbm.at[wid])

    return seg_sum_kernel(ei2d, ev2d)


def _tc_body(h_ref, sp_ref, norm_ref, w_ref, b_ref, o_ref):
    ones = jnp.ones((_NW, 1), jnp.float32)
    s = jax.lax.dot_general(
        sp_ref[...], ones, (((0,), (0,)), ((), ())),
        preferred_element_type=jnp.float32)
    x = h_ref[...] * s
    y = jax.lax.dot_general(
        x, w_ref[...], (((1,), (1,)), ((), ())),
        preferred_element_type=jnp.float32)
    o_ref[...] = (y + b_ref[...]) * norm_ref[...]


def _tc_apply(h, s_part, norm, w, b2d):
    return pl.pallas_call(
        _tc_body,
        out_shape=jax.ShapeDtypeStruct((_N, _D_OUT), jnp.float32),
    )(h, s_part, norm, w, b2d)


def kernel(h, e_h, norm, edge_index, W, b):
    ev = e_h.reshape(1, _E)
    s_part = _sc_segment_sum(edge_index, ev)
    return _tc_apply(h, s_part, norm, W, b.reshape(1, _D_OUT))
